# Initial kernel scaffold; baseline (speedup 1.0000x reference)
import jax
import jax.numpy as jnp
from jax.experimental import pallas as pl


def kernel(x_node, x_link, edge_index_nl, edge_index_ln, batch):
    raise NotImplementedError("write your pallas kernel here")


# speedup vs baseline: 4.0149x; 4.0149x over previous
import functools

import jax
import jax.numpy as jnp
from jax import lax
from jax.experimental import pallas as pl
from jax.experimental.pallas import tpu as pltpu
from jax.experimental.pallas import tpu_sc as plsc

N = 50000
D = 128
NP = 50176
CS = 12544
ACC_R = 12800
E = 500000
EP = 507904
EB = 512
NB = EP // (16 * EB)
TS = EP // 16
FL = 128
NG = 16

_SDS = jax.ShapeDtypeStruct


def _zero_rowbuf(rowbuf):
    def zb(r, carry):
        for q in range(8):
            rowbuf[r, pl.ds(16 * q, 16)] = jnp.zeros((16,), jnp.float32)
        return carry
    lax.fori_loop(0, FL, zb, 0)


def _agg_body(table, src, dst, out, acc, cnt, esrc, edst, glist, dlist,
              rowbuf, nbuf, cbuf, ones_f, gsem):
    core = lax.axis_index("c")
    sub = lax.axis_index("s")
    lane = lax.iota(jnp.int32, 16)
    zero16 = jnp.zeros((16,), jnp.int32)

    for q in range(8):
        ones_f[pl.ds(16 * q, 16)] = jnp.ones((16,), jnp.float32)

    def flush(cur, tail):
        gi = glist.at[0]
        di = dlist.at[0]
        pltpu.async_copy(table.at[gi], rowbuf, gsem).wait()
        pltpu.sync_copy(rowbuf, acc.at[di], add=True)
        pltpu.sync_copy(ones_f, cnt.at[di], add=True)
        if tail:
            return jnp.int32(0)
        spill = cur - FL
        vg = glist[1, pl.ds(0, 16)]
        vd = dlist[1, pl.ds(0, 16)]
        mm = lane < spill
        plsc.store_scatter(glist, [zero16, lane], vg, mask=mm)
        plsc.store_scatter(dlist, [zero16, lane], vd, mask=mm)
        return spill

    for c in range(2):
        cid = core * 2 + c
        lo = cid * CS

        _zero_rowbuf(rowbuf)
        rz = 800 * sub
        for j2 in range(6):
            pltpu.sync_copy(rowbuf, acc.at[pl.ds(rz + FL * j2, FL)])
            pltpu.sync_copy(rowbuf.at[0], cnt.at[pl.ds(rz + FL * j2, FL)])
        pltpu.sync_copy(rowbuf.at[pl.ds(0, 32)], acc.at[pl.ds(rz + 768, 32)])
        pltpu.sync_copy(rowbuf.at[0, pl.ds(0, 32)], cnt.at[pl.ds(rz + 768, 32)])
        plsc.subcore_barrier()

        def vreg_body(k, cur):
            s = esrc[pl.ds(16 * k, 16)]
            d = edst[pl.ds(16 * k, 16)]
            loc = d - lo
            m = (loc >= 0) & (loc < CS)
            inc = plsc.cumsum(jnp.where(m, 1, 0).astype(jnp.int32))
            pos = cur + inc - 1
            rowi = lax.shift_right_logical(pos, 7)
            coli = lax.bitwise_and(pos, 127)
            plsc.store_scatter(glist, [rowi, coli], s, mask=m)
            plsc.store_scatter(dlist, [rowi, coli], loc, mask=m)
            cur = cur + inc[15]
            return lax.cond(cur >= FL, lambda t: flush(t, False),
                            lambda t: t, cur)

        def batch_body(j, cur):
            eb = sub * TS + EB * j
            pltpu.sync_copy(src.at[pl.ds(eb, EB)], esrc)
            pltpu.sync_copy(dst.at[pl.ds(eb, EB)], edst)
            return lax.fori_loop(0, EB // 16, vreg_body, cur)

        cursor = lax.fori_loop(0, NB, batch_body, jnp.int32(0))

        def pad_body(p, carry):
            pp = 16 * p + lane
            mm = pp >= cursor
            dsrc = pp * 157 + sub * 16
            dloc = CS + lax.bitwise_and(pp + sub * 8, 255)
            plsc.store_scatter(glist, [zero16, pp], dsrc, mask=mm)
            plsc.store_scatter(dlist, [zero16, pp], dloc, mask=mm)
            return carry
        lax.fori_loop(0, 8, pad_body, 0)
        flush(jnp.int32(FL), True)
        plsc.subcore_barrier()

        nb_base = 784 * sub

        def norm_body(b, carry):
            rb = nb_base + 16 * b
            pltpu.sync_copy(acc.at[pl.ds(rb, 16)], nbuf)
            pltpu.sync_copy(cnt.at[pl.ds(rb, 16)], cbuf)
            cv = cbuf[pl.ds(0, 16)]
            iv = 1.0 / jnp.maximum(cv, 1.0)
            for r in range(16):
                sc = iv[r]
                for q in range(8):
                    v = nbuf[r, pl.ds(16 * q, 16)]
                    nbuf[r, pl.ds(16 * q, 16)] = jnp.maximum(v * sc, 0.0)
            pltpu.sync_copy(nbuf, out.at[pl.ds(lo + rb, 16)])
            return carry
        lax.fori_loop(0, 49, norm_body, 0)
        plsc.subcore_barrier()


_agg = pl.kernel(
    _agg_body,
    out_type=_SDS((NP, D), jnp.float32),
    mesh=plsc.VectorSubcoreMesh(core_axis_name="c", subcore_axis_name="s"),
    scratch_types=[
        pltpu.VMEM_SHARED((ACC_R, D), jnp.float32),
        pltpu.VMEM_SHARED((ACC_R,), jnp.float32),
        pltpu.VMEM((EB,), jnp.int32),
        pltpu.VMEM((EB,), jnp.int32),
        pltpu.VMEM((2, FL), jnp.int32),
        pltpu.VMEM((2, FL), jnp.int32),
        pltpu.VMEM((FL, D), jnp.float32),
        pltpu.VMEM((16, D), jnp.float32),
        pltpu.VMEM((16,), jnp.float32),
        pltpu.VMEM((FL,), jnp.float32),
        pltpu.SemaphoreType.DMA,
    ],
    compiler_params=pltpu.CompilerParams(needs_layout_passes=False),
    name="hgnn_mean_agg",
)

POOL_ROWS = NP // 32


def _pool_body(x, b, psum, pcnt, pacc, pcacc, rbuf, bbuf, cb2, onesp):
    core = lax.axis_index("c")
    sub = lax.axis_index("s")
    w = core * 16 + sub

    for q in range(7):
        onesp[pl.ds(16 * q, 16)] = jnp.ones((16,), jnp.float32)

    def zb(r, carry):
        for q in range(8):
            rbuf[r, pl.ds(16 * q, 16)] = jnp.zeros((16,), jnp.float32)
        return carry
    lax.fori_loop(0, 32, zb, 0)

    @pl.when(sub == 0)
    def _():
        pltpu.sync_copy(rbuf.at[pl.ds(0, 32)], pacc)
        pltpu.sync_copy(rbuf.at[0, pl.ds(0, 32)], pcacc)
    plsc.subcore_barrier()

    def bb(i, carry):
        st = w * POOL_ROWS + 112 * i
        pltpu.sync_copy(x.at[pl.ds(st, 112)], rbuf)
        pltpu.sync_copy(b.at[pl.ds(st, 112)], bbuf)
        pltpu.sync_copy(rbuf, pacc.at[bbuf], add=True)
        pltpu.sync_copy(onesp, pcacc.at[bbuf], add=True)
        return carry
    lax.fori_loop(0, POOL_ROWS // 112, bb, 0)
    plsc.subcore_barrier()

    @pl.when(sub == 0)
    def _():
        pltpu.sync_copy(pcacc, cb2)
        pltpu.sync_copy(pacc.at[pl.ds(0, 16)], rbuf.at[pl.ds(0, 16)])
        cv = cb2[pl.ds(0, 16)]
        for k in range(16):
            vv = jnp.zeros((16,), jnp.float32) + cv[k]
            for q in range(8):
                rbuf[16 + k, pl.ds(16 * q, 16)] = vv
        pltpu.sync_copy(rbuf.at[pl.ds(0, 16)], psum.at[core])
        pltpu.sync_copy(rbuf.at[pl.ds(16, 16)], pcnt.at[core])


_pool = pl.kernel(
    _pool_body,
    out_type=[_SDS((2, NG, D), jnp.float32), _SDS((2, NG, D), jnp.float32)],
    mesh=plsc.VectorSubcoreMesh(core_axis_name="c", subcore_axis_name="s"),
    scratch_types=[
        pltpu.VMEM_SHARED((32, D), jnp.float32),
        pltpu.VMEM_SHARED((32,), jnp.float32),
        pltpu.VMEM((112, D), jnp.float32),
        pltpu.VMEM((112,), jnp.int32),
        pltpu.VMEM((32,), jnp.float32),
        pltpu.VMEM((112,), jnp.float32),
    ],
    compiler_params=pltpu.CompilerParams(needs_layout_passes=False),
    name="hgnn_pool",
)


def _comb_body(s_ref, c_ref, o_ref):
    o_ref[...] = (s_ref[0] + s_ref[1]) / jnp.maximum(c_ref[0] + c_ref[1], 1.0)


def _combine(psum, pcnt):
    return pl.pallas_call(
        _comb_body,
        out_shape=_SDS((NG, D), jnp.float32),
    )(psum, pcnt)


def kernel(x_node, x_link, edge_index_nl, edge_index_ln, batch):
    npad = EP - E
    pad_src = ((jnp.arange(npad, dtype=jnp.int32) * 7919) % N).astype(jnp.int32)
    pad_dst = (N + jnp.arange(npad, dtype=jnp.int32) % (NP - N)).astype(jnp.int32)
    nl_s = jnp.concatenate([edge_index_nl[0], pad_src])
    nl_d = jnp.concatenate([edge_index_nl[1], pad_dst])
    ln_s = jnp.concatenate([edge_index_ln[0], pad_src])
    ln_d = jnp.concatenate([edge_index_ln[1], pad_dst])
    xp = jnp.concatenate([x_node, jnp.zeros((NP - N, D), jnp.float32)])
    link1 = _agg(xp, nl_s, nl_d)
    node2 = _agg(link1, ln_s, ln_d)
    link3 = _agg(node2, nl_s, nl_d)
    bpad = (NG + jnp.arange(NP - N, dtype=jnp.int32) % NG).astype(jnp.int32)
    bp = jnp.concatenate([batch, bpad])
    psum, pcnt = _pool(link3, bp)
    return _combine(psum, pcnt)
